# baseline (speedup 1.0000x reference)
import functools

import jax
import jax.numpy as jnp
from jax import lax
from jax.experimental import pallas as pl
from jax.experimental.pallas import tpu as pltpu
from jax.experimental.pallas import tpu_sc as plsc

_NC = 2
_NS = 16
_L = 16
_NW = _NC * _NS

_TPAD = 1024


def _sc_gather(rg_pad, ig_pad, t_flat, n):
    chunk = n // _NW
    mesh = plsc.VectorSubcoreMesh(core_axis_name="c", subcore_axis_name="s")

    @functools.partial(
        pl.kernel,
        out_type=(
            jax.ShapeDtypeStruct((n,), jnp.float32),
            jax.ShapeDtypeStruct((n,), jnp.float32),
            jax.ShapeDtypeStruct((n,), jnp.int32),
        ),
        mesh=mesh,
        compiler_params=pltpu.CompilerParams(needs_layout_passes=False),
        scratch_types=[
            pltpu.VMEM((_TPAD,), jnp.float32),
            pltpu.VMEM((_TPAD,), jnp.float32),
            pltpu.VMEM((chunk,), jnp.int32),
            pltpu.VMEM((chunk,), jnp.float32),
            pltpu.VMEM((chunk,), jnp.float32),
            pltpu.SemaphoreType.DMA,
        ],
    )
    def gather_kernel(rg_hbm, ig_hbm, t_hbm, outr_hbm, outi_hbm, tout_hbm,
                      rg_v, ig_v, t_v, outr_v, outi_v, sem):
        wid = lax.axis_index("s") * _NC + lax.axis_index("c")
        base = wid * chunk
        cp1 = pltpu.async_copy(rg_hbm, rg_v, sem)
        cp2 = pltpu.async_copy(ig_hbm, ig_v, sem)
        cp3 = pltpu.async_copy(t_hbm.at[pl.ds(base, chunk)], t_v, sem)
        cp1.wait()
        cp2.wait()
        cp3.wait()
        for i in range(chunk // _L):
            idx = t_v[pl.ds(i * _L, _L)]
            outr_v[pl.ds(i * _L, _L)] = plsc.load_gather(rg_v, [idx])
            outi_v[pl.ds(i * _L, _L)] = plsc.load_gather(ig_v, [idx])
        cp4 = pltpu.async_copy(outr_v, outr_hbm.at[pl.ds(base, chunk)], sem)
        cp5 = pltpu.async_copy(outi_v, outi_hbm.at[pl.ds(base, chunk)], sem)
        cp6 = pltpu.async_copy(t_v, tout_hbm.at[pl.ds(base, chunk)], sem)
        cp4.wait()
        cp5.wait()
        cp6.wait()

    return gather_kernel(rg_pad, ig_pad, t_flat)


def _noise_body(gr_ref, gi_ref, real_ref, rn_ref, imag_ref, inz_ref,
                outr_ref, outi_ref, rn_out_ref, inz_out_ref):
    i = pl.program_id(0)
    rows = real_ref.shape[0]
    gr = gr_ref[pl.ds(i * rows, rows), :]
    rn = rn_ref[...]
    outr_ref[...] = jnp.sqrt(gr) * real_ref[...] + jnp.sqrt(1.0 - gr) * rn
    rn_out_ref[...] = rn
    gi = gi_ref[pl.ds(i * rows, rows), :]
    inz = inz_ref[...]
    outi_ref[...] = jnp.sqrt(gi) * imag_ref[...] + jnp.sqrt(1.0 - gi) * inz
    inz_out_ref[...] = inz


def _tc_noise(gr, gi, real2, rn2, imag2, inz2, block_rows):
    n, d = real2.shape
    grid = (n // block_rows,)
    row_spec = pl.BlockSpec((block_rows, d), lambda i: (i, 0))
    g_spec = pl.BlockSpec((n, 1), lambda i: (0, 0))
    arr = jax.ShapeDtypeStruct((n, d), jnp.float32)
    return pl.pallas_call(
        _noise_body,
        grid=grid,
        in_specs=[g_spec, g_spec, row_spec, row_spec, row_spec, row_spec],
        out_specs=[row_spec, row_spec, row_spec, row_spec],
        out_shape=(arr, arr, arr, arr),
    )(gr, gi, real2, rn2, imag2, inz2)


def kernel(real, imag, real_gamma, imag_gamma, t, real_noise, imag_noise):
    b, s, d = real.shape
    n = b * s
    tlen = real_gamma.shape[0]

    rg_pad = jnp.pad(real_gamma, (0, _TPAD - tlen))
    ig_pad = jnp.pad(imag_gamma, (0, _TPAD - tlen))
    t_flat = t.reshape(n).astype(jnp.int32)

    gr, gi, t_out = _sc_gather(rg_pad, ig_pad, t_flat, n)

    real_noisy, imag_noisy, rn_out, inz_out = _tc_noise(
        gr.reshape(n, 1), gi.reshape(n, 1),
        real.reshape(n, d), real_noise.reshape(n, d),
        imag.reshape(n, d), imag_noise.reshape(n, d),
        block_rows=512,
    )
    return (real_noisy.reshape(b, s, d), rn_out.reshape(b, s, d),
            imag_noisy.reshape(b, s, d), inz_out.reshape(b, s, d),
            t_out.reshape(b, s).astype(t.dtype))


# speedup vs baseline: 1.0090x; 1.0090x over previous
import functools

import jax
import jax.numpy as jnp
from jax import lax
from jax.experimental import pallas as pl
from jax.experimental.pallas import tpu as pltpu
from jax.experimental.pallas import tpu_sc as plsc

_NC = 2
_NS = 16
_L = 16
_NW = _NC * _NS

_TPAD = 1024


def _sc_gather(rg, ig, t_flat, n):
    chunk = n // _NW
    tlen = rg.shape[0]
    mesh = plsc.VectorSubcoreMesh(core_axis_name="c", subcore_axis_name="s")

    @functools.partial(
        pl.kernel,
        out_type=(
            jax.ShapeDtypeStruct((n,), jnp.float32),
            jax.ShapeDtypeStruct((n,), jnp.float32),
            jax.ShapeDtypeStruct((n,), jnp.int32),
        ),
        mesh=mesh,
        compiler_params=pltpu.CompilerParams(needs_layout_passes=False),
        scratch_types=[
            pltpu.VMEM((_TPAD,), jnp.float32),
            pltpu.VMEM((_TPAD,), jnp.float32),
            pltpu.VMEM((chunk,), jnp.int32),
            pltpu.VMEM((chunk,), jnp.float32),
            pltpu.VMEM((chunk,), jnp.float32),
            pltpu.SemaphoreType.DMA,
        ],
    )
    def gather_kernel(rg_hbm, ig_hbm, t_hbm, outr_hbm, outi_hbm, tout_hbm,
                      rg_v, ig_v, t_v, outr_v, outi_v, sem):
        wid = lax.axis_index("s") * _NC + lax.axis_index("c")
        base = wid * chunk
        cp1 = pltpu.async_copy(rg_hbm, rg_v.at[pl.ds(0, tlen)], sem)
        cp2 = pltpu.async_copy(ig_hbm, ig_v.at[pl.ds(0, tlen)], sem)
        cp3 = pltpu.async_copy(t_hbm.at[pl.ds(base, chunk)], t_v, sem)
        cp1.wait()
        cp2.wait()
        cp3.wait()
        for i in range(chunk // _L):
            idx = t_v[pl.ds(i * _L, _L)]
            outr_v[pl.ds(i * _L, _L)] = plsc.load_gather(rg_v, [idx])
            outi_v[pl.ds(i * _L, _L)] = plsc.load_gather(ig_v, [idx])
        cp4 = pltpu.async_copy(outr_v, outr_hbm.at[pl.ds(base, chunk)], sem)
        cp5 = pltpu.async_copy(outi_v, outi_hbm.at[pl.ds(base, chunk)], sem)
        cp6 = pltpu.async_copy(t_v, tout_hbm.at[pl.ds(base, chunk)], sem)
        cp4.wait()
        cp5.wait()
        cp6.wait()

    return gather_kernel(rg, ig, t_flat)


def _noise_body(gr_ref, gi_ref, real_ref, rn_ref, imag_ref, inz_ref,
                outr_ref, outi_ref, rn_out_ref, inz_out_ref):
    gr = gr_ref[...]
    rn = rn_ref[...]
    outr_ref[...] = jnp.sqrt(gr) * real_ref[...] + jnp.sqrt(1.0 - gr) * rn
    rn_out_ref[...] = rn
    gi = gi_ref[...]
    inz = inz_ref[...]
    outi_ref[...] = jnp.sqrt(gi) * imag_ref[...] + jnp.sqrt(1.0 - gi) * inz
    inz_out_ref[...] = inz


def _tc_noise(gr, gi, real2, rn2, imag2, inz2, block_rows):
    n, d = real2.shape
    grid = (n // block_rows,)
    row_spec = pl.BlockSpec((block_rows, d), lambda i: (i, 0))
    g_spec = pl.BlockSpec((block_rows, 1), lambda i: (i, 0))
    arr = jax.ShapeDtypeStruct((n, d), jnp.float32)
    return pl.pallas_call(
        _noise_body,
        grid=grid,
        in_specs=[g_spec, g_spec, row_spec, row_spec, row_spec, row_spec],
        out_specs=[row_spec, row_spec, row_spec, row_spec],
        out_shape=(arr, arr, arr, arr),
    )(gr, gi, real2, rn2, imag2, inz2)


def kernel(real, imag, real_gamma, imag_gamma, t, real_noise, imag_noise):
    b, s, d = real.shape
    n = b * s

    t_flat = t.reshape(n).astype(jnp.int32)

    gr, gi, t_out = _sc_gather(real_gamma, imag_gamma, t_flat, n)

    real_noisy, imag_noisy, rn_out, inz_out = _tc_noise(
        gr.reshape(n, 1), gi.reshape(n, 1),
        real.reshape(n, d), real_noise.reshape(n, d),
        imag.reshape(n, d), imag_noise.reshape(n, d),
        block_rows=512,
    )
    return (real_noisy.reshape(b, s, d), rn_out.reshape(b, s, d),
            imag_noisy.reshape(b, s, d), inz_out.reshape(b, s, d),
            t_out.reshape(b, s).astype(t.dtype))


# speedup vs baseline: 1.0950x; 1.0853x over previous
import functools

import jax
import jax.numpy as jnp
from jax import lax
from jax.experimental import pallas as pl
from jax.experimental.pallas import tpu as pltpu
from jax.experimental.pallas import tpu_sc as plsc

_NC = 2
_NS = 16
_L = 16
_NW = _NC * _NS

_TPAD = 1024


def _sc_gather(rg, ig, t_flat, n):
    chunk = n // _NW
    tlen = rg.shape[0]
    mesh = plsc.VectorSubcoreMesh(core_axis_name="c", subcore_axis_name="s")

    @functools.partial(
        pl.kernel,
        out_type=(
            jax.ShapeDtypeStruct((n,), jnp.float32),
            jax.ShapeDtypeStruct((n,), jnp.float32),
            jax.ShapeDtypeStruct((n,), jnp.int32),
        ),
        mesh=mesh,
        compiler_params=pltpu.CompilerParams(needs_layout_passes=False),
        scratch_types=[
            pltpu.VMEM((_TPAD,), jnp.float32),
            pltpu.VMEM((_TPAD,), jnp.float32),
            pltpu.VMEM((chunk,), jnp.int32),
            pltpu.VMEM((chunk,), jnp.float32),
            pltpu.VMEM((chunk,), jnp.float32),
            pltpu.SemaphoreType.DMA,
        ],
    )
    def gather_kernel(rg_hbm, ig_hbm, t_hbm, outr_hbm, outi_hbm, tout_hbm,
                      rg_v, ig_v, t_v, outr_v, outi_v, sem):
        wid = lax.axis_index("s") * _NC + lax.axis_index("c")
        base = wid * chunk
        cp1 = pltpu.async_copy(rg_hbm, rg_v.at[pl.ds(0, tlen)], sem)
        cp2 = pltpu.async_copy(ig_hbm, ig_v.at[pl.ds(0, tlen)], sem)
        cp3 = pltpu.async_copy(t_hbm.at[pl.ds(base, chunk)], t_v, sem)
        cp1.wait()
        cp2.wait()
        cp3.wait()
        for i in range(chunk // _L):
            idx = t_v[pl.ds(i * _L, _L)]
            outr_v[pl.ds(i * _L, _L)] = plsc.load_gather(rg_v, [idx])
            outi_v[pl.ds(i * _L, _L)] = plsc.load_gather(ig_v, [idx])
        cp4 = pltpu.async_copy(outr_v, outr_hbm.at[pl.ds(base, chunk)], sem)
        cp5 = pltpu.async_copy(outi_v, outi_hbm.at[pl.ds(base, chunk)], sem)
        cp6 = pltpu.async_copy(t_v, tout_hbm.at[pl.ds(base, chunk)], sem)
        cp4.wait()
        cp5.wait()
        cp6.wait()

    return gather_kernel(rg, ig, t_flat)


def _noise_body(gr_ref, gi_ref, real_ref, rn_ref, imag_ref, inz_ref,
                outr_ref, outi_ref, rn_out_ref, inz_out_ref):
    rows = real_ref.shape[0]
    gr = gr_ref[...].reshape(1, rows).T
    rn = rn_ref[...]
    outr_ref[...] = jnp.sqrt(gr) * real_ref[...] + jnp.sqrt(1.0 - gr) * rn
    rn_out_ref[...] = rn
    gi = gi_ref[...].reshape(1, rows).T
    inz = inz_ref[...]
    outi_ref[...] = jnp.sqrt(gi) * imag_ref[...] + jnp.sqrt(1.0 - gi) * inz
    inz_out_ref[...] = inz


def _tc_noise(gr, gi, real2, rn2, imag2, inz2, block_rows):
    n, d = real2.shape
    grid = (n // block_rows,)
    row_spec = pl.BlockSpec((block_rows, d), lambda i: (i, 0))
    g_spec = pl.BlockSpec((block_rows,), lambda i: (i,))
    arr = jax.ShapeDtypeStruct((n, d), jnp.float32)
    return pl.pallas_call(
        _noise_body,
        grid=grid,
        in_specs=[g_spec, g_spec, row_spec, row_spec, row_spec, row_spec],
        out_specs=[row_spec, row_spec, row_spec, row_spec],
        out_shape=(arr, arr, arr, arr),
    )(gr, gi, real2, rn2, imag2, inz2)


def kernel(real, imag, real_gamma, imag_gamma, t, real_noise, imag_noise):
    b, s, d = real.shape
    n = b * s

    t_flat = t.reshape(n).astype(jnp.int32)

    gr, gi, t_out = _sc_gather(real_gamma, imag_gamma, t_flat, n)

    real_noisy, imag_noisy, rn_out, inz_out = _tc_noise(
        gr, gi,
        real.reshape(n, d), real_noise.reshape(n, d),
        imag.reshape(n, d), imag_noise.reshape(n, d),
        block_rows=512,
    )
    return (real_noisy.reshape(b, s, d), rn_out.reshape(b, s, d),
            imag_noisy.reshape(b, s, d), inz_out.reshape(b, s, d),
            t_out.reshape(b, s).astype(t.dtype))


# speedup vs baseline: 1.1007x; 1.0051x over previous
import functools

import jax
import jax.numpy as jnp
from jax import lax
from jax.experimental import pallas as pl
from jax.experimental.pallas import tpu as pltpu
from jax.experimental.pallas import tpu_sc as plsc

_NC = 2
_NS = 16
_L = 16
_NW = _NC * _NS

_TPAD = 1024


def _sc_gather(rg, ig, t_flat, n):
    chunk = n // _NW
    tlen = rg.shape[0]
    mesh = plsc.VectorSubcoreMesh(core_axis_name="c", subcore_axis_name="s")

    @functools.partial(
        pl.kernel,
        out_type=(
            jax.ShapeDtypeStruct((n,), jnp.float32),
            jax.ShapeDtypeStruct((n,), jnp.float32),
        ),
        mesh=mesh,
        compiler_params=pltpu.CompilerParams(needs_layout_passes=False),
        scratch_types=[
            pltpu.VMEM((_TPAD,), jnp.float32),
            pltpu.VMEM((_TPAD,), jnp.float32),
            pltpu.VMEM((chunk,), jnp.int32),
            pltpu.VMEM((chunk,), jnp.float32),
            pltpu.VMEM((chunk,), jnp.float32),
            pltpu.SemaphoreType.DMA,
        ],
    )
    def gather_kernel(rg_hbm, ig_hbm, t_hbm, outr_hbm, outi_hbm,
                      rg_v, ig_v, t_v, outr_v, outi_v, sem):
        wid = lax.axis_index("s") * _NC + lax.axis_index("c")
        base = wid * chunk
        cp1 = pltpu.async_copy(rg_hbm, rg_v.at[pl.ds(0, tlen)], sem)
        cp2 = pltpu.async_copy(ig_hbm, ig_v.at[pl.ds(0, tlen)], sem)
        cp3 = pltpu.async_copy(t_hbm.at[pl.ds(base, chunk)], t_v, sem)
        cp1.wait()
        cp2.wait()
        cp3.wait()
        for i in range(chunk // _L):
            idx = t_v[pl.ds(i * _L, _L)]
            outr_v[pl.ds(i * _L, _L)] = plsc.load_gather(rg_v, [idx])
            outi_v[pl.ds(i * _L, _L)] = plsc.load_gather(ig_v, [idx])
        cp4 = pltpu.async_copy(outr_v, outr_hbm.at[pl.ds(base, chunk)], sem)
        cp5 = pltpu.async_copy(outi_v, outi_hbm.at[pl.ds(base, chunk)], sem)
        cp4.wait()
        cp5.wait()

    return gather_kernel(rg, ig, t_flat)


def _noise_body(gr_ref, gi_ref, real_ref, rn_ref, imag_ref, inz_ref,
                outr_ref, outi_ref, rn_out_ref, inz_out_ref):
    rows = real_ref.shape[0]
    gr = gr_ref[...].reshape(1, rows).T
    rn = rn_ref[...]
    outr_ref[...] = jnp.sqrt(gr) * real_ref[...] + jnp.sqrt(1.0 - gr) * rn
    rn_out_ref[...] = rn
    gi = gi_ref[...].reshape(1, rows).T
    inz = inz_ref[...]
    outi_ref[...] = jnp.sqrt(gi) * imag_ref[...] + jnp.sqrt(1.0 - gi) * inz
    inz_out_ref[...] = inz


def _tc_noise(gr, gi, real2, rn2, imag2, inz2, block_rows):
    n, d = real2.shape
    grid = (n // block_rows,)
    row_spec = pl.BlockSpec((block_rows, d), lambda i: (i, 0))
    g_spec = pl.BlockSpec((block_rows,), lambda i: (i,))
    arr = jax.ShapeDtypeStruct((n, d), jnp.float32)
    return pl.pallas_call(
        _noise_body,
        grid=grid,
        in_specs=[g_spec, g_spec, row_spec, row_spec, row_spec, row_spec],
        out_specs=[row_spec, row_spec, row_spec, row_spec],
        out_shape=(arr, arr, arr, arr),
    )(gr, gi, real2, rn2, imag2, inz2)


def kernel(real, imag, real_gamma, imag_gamma, t, real_noise, imag_noise):
    b, s, d = real.shape
    n = b * s

    t_flat = t.reshape(n).astype(jnp.int32)

    gr, gi = _sc_gather(real_gamma, imag_gamma, t_flat, n)

    real_noisy, imag_noisy, rn_out, inz_out = _tc_noise(
        gr, gi,
        real.reshape(n, d), real_noise.reshape(n, d),
        imag.reshape(n, d), imag_noise.reshape(n, d),
        block_rows=512,
    )
    return (real_noisy.reshape(b, s, d), rn_out.reshape(b, s, d),
            imag_noisy.reshape(b, s, d), inz_out.reshape(b, s, d), t)
